# baseline (speedup 1.0000x reference)
import jax
import jax.numpy as jnp
from jax import lax
from jax.experimental import pallas as pl
from jax.experimental.pallas import tpu as pltpu
from jax.experimental.pallas import tpu_sc as plsc

N_NODES = 10000
N_EDGES = 320000
D_IN = 128
D_HID = 32
N_CLASSES = 16

_LANES = 16
_CHUNK = 4000
_N_TILES = 32


def _make_sc_propagate(n_feat, dims):
  tiles_per_group = n_feat // dims
  n_groups = _N_TILES // tiles_per_group
  e_grp = N_EDGES // n_groups
  n_chunks = e_grp // _CHUNK
  assert n_chunks * _CHUNK == e_grp

  def zero_columns(outs):
    zero = jnp.zeros((_LANES,), jnp.float32)

    @plsc.parallel_loop(0, N_NODES, _LANES)
    def _(i):
      sl = pl.ds(i, _LANES)
      for o in outs:
        o[sl] = zero

  def edge_loop(tabs, outs, sv, dv, wv):
    @plsc.parallel_loop(0, _CHUNK, _LANES, unroll=8)
    def _(i):
      sl = pl.ds(i, _LANES)
      s_idx = sv[sl]
      d_idx = dv[sl]
      w = wv[sl]
      for t in range(dims):
        g = plsc.load_gather(tabs[t], [s_idx]) * w
        plsc.addupdate_scatter(outs[t], [d_idx], g)

  def start_chunk(egrp, k, src, dst, w, buf):
    off = egrp * e_grp + k * _CHUNK
    sv, dv, wv, sem = buf
    return (
        pltpu.async_copy(src.at[pl.ds(off, _CHUNK)], sv, sem),
        pltpu.async_copy(dst.at[pl.ds(off, _CHUNK)], dv, sem),
        pltpu.async_copy(w.at[pl.ds(off, _CHUNK)], wv, sem),
    )

  def body(ht, src, dst, w, out, *scratch):
    tabs = scratch[0:dims]
    outs = scratch[dims:2 * dims]
    bufs = (scratch[2 * dims:2 * dims + 4], scratch[2 * dims + 4:2 * dims + 8])
    tile = lax.axis_index("c") * 16 + lax.axis_index("s")
    egrp = tile // tiles_per_group
    dbase = (tile % tiles_per_group) * dims
    for t in range(dims):
      pltpu.sync_copy(ht.at[dbase + t], tabs[t])
    zero_columns(outs)
    descs = start_chunk(egrp, 0, src, dst, w, bufs[0])
    for k in range(n_chunks):
      for d in descs:
        d.wait()
      cur = bufs[k % 2]
      if k + 1 < n_chunks:
        descs = start_chunk(egrp, k + 1, src, dst, w, bufs[(k + 1) % 2])
      edge_loop(tabs, outs, cur[0], cur[1], cur[2])
    for t in range(dims):
      pltpu.sync_copy(outs[t], out.at[egrp, dbase + t])

  mesh = plsc.VectorSubcoreMesh(core_axis_name="c", subcore_axis_name="s")
  return pl.kernel(
      body,
      out_type=jax.ShapeDtypeStruct((n_groups, n_feat, N_NODES), jnp.float32),
      mesh=mesh,
      compiler_params=pltpu.CompilerParams(needs_layout_passes=False),
      scratch_types=[
          *[pltpu.VMEM((N_NODES,), jnp.float32) for _ in range(2 * dims)],
          pltpu.VMEM((_CHUNK,), jnp.int32),
          pltpu.VMEM((_CHUNK,), jnp.int32),
          pltpu.VMEM((_CHUNK,), jnp.float32),
          pltpu.SemaphoreType.DMA,
          pltpu.VMEM((_CHUNK,), jnp.int32),
          pltpu.VMEM((_CHUNK,), jnp.int32),
          pltpu.VMEM((_CHUNK,), jnp.float32),
          pltpu.SemaphoreType.DMA,
      ],
  )


def _mm1_body(x_ref, w_ref, out_ref):
  out_ref[...] = lax.dot_general(
      w_ref[...], x_ref[...], (((0,), (1,)), ((), ())),
      preferred_element_type=jnp.float32)


def _mid_body(p_ref, b1_ref, w2_ref, out_ref):
  h = p_ref[0] + p_ref[1] + p_ref[2] + p_ref[3]
  h = jnp.maximum(h + b1_ref[...], 0.0)
  out_ref[...] = lax.dot_general(
      w2_ref[...], h, (((0,), (0,)), ((), ())),
      preferred_element_type=jnp.float32)


def _final_body(p_ref, b2_ref, eye_ref, out_ref):
  g = p_ref[0]
  for i in range(1, p_ref.shape[0]):
    g = g + p_ref[i]
  logits = lax.dot_general(
      g, eye_ref[...], (((0,), (0,)), ((), ())),
      preferred_element_type=jnp.float32)
  z = logits + b2_ref[0][None, :]
  m = jnp.max(z, axis=1, keepdims=True)
  lse = jnp.log(jnp.sum(jnp.exp(z - m), axis=1, keepdims=True)) + m
  out_ref[...] = z - lse


@jax.jit
def kernel(x, edge_index, edge_weight, W1, b1, W2, b2):
  src = edge_index[0].astype(jnp.int32)
  dst = edge_index[1].astype(jnp.int32)
  w = edge_weight.astype(jnp.float32)

  ht = pl.pallas_call(
      _mm1_body,
      out_shape=jax.ShapeDtypeStruct((D_HID, N_NODES), jnp.float32),
  )(x, W1)

  p1 = _make_sc_propagate(D_HID, 4)(ht, src, dst, w)

  gt = pl.pallas_call(
      _mid_body,
      out_shape=jax.ShapeDtypeStruct((N_CLASSES, N_NODES), jnp.float32),
  )(p1, b1.reshape(D_HID, 1), W2)

  p2 = _make_sc_propagate(N_CLASSES, 4)(gt, src, dst, w)

  out = pl.pallas_call(
      _final_body,
      out_shape=jax.ShapeDtypeStruct((N_NODES, N_CLASSES), jnp.float32),
  )(p2, b2.reshape(1, N_CLASSES), jnp.eye(N_CLASSES, dtype=jnp.float32))
  return out


# speedup vs baseline: 1.1413x; 1.1413x over previous
import jax
import jax.numpy as jnp
from jax import lax
from jax.experimental import pallas as pl
from jax.experimental.pallas import tpu as pltpu
from jax.experimental.pallas import tpu_sc as plsc

N_NODES = 10000
N_EDGES = 320000
D_IN = 128
D_HID = 32
N_CLASSES = 16

_LANES = 16
_CHUNK = 4000
_N_TILES = 32

_HI_MASK = -65536


def _pack_pairs(h):
  k = h.shape[0] // 2
  r = lax.bitcast_convert_type(h, jnp.int32) + jnp.int32(0x8000)
  hi = jnp.bitwise_and(r[:k], jnp.int32(_HI_MASK))
  lo = lax.shift_right_logical(r[k:], 16)
  return jnp.bitwise_or(hi, lo)


def _make_sc_propagate(n_feat, dims):
  n_pk = dims // 2
  tiles_per_group = n_feat // dims
  n_groups = _N_TILES // tiles_per_group
  e_grp = N_EDGES // n_groups
  n_chunks = e_grp // _CHUNK
  assert n_chunks * _CHUNK == e_grp

  def zero_columns(outs):
    zero = jnp.zeros((_LANES,), jnp.float32)

    @plsc.parallel_loop(0, N_NODES, _LANES)
    def _(i):
      sl = pl.ds(i, _LANES)
      for o in outs:
        o[sl] = zero

  def edge_loop(tabs, outs, sv, dv, wv):
    @plsc.parallel_loop(0, _CHUNK, _LANES, unroll=4)
    def _(i):
      sl = pl.ds(i, _LANES)
      s_idx = sv[sl]
      d_idx = dv[sl]
      w = wv[sl]
      for t in range(n_pk):
        g = plsc.load_gather(tabs[t], [s_idx])
        hi = lax.bitcast_convert_type(jnp.bitwise_and(g, jnp.int32(_HI_MASK)),
                                      jnp.float32)
        lo = lax.bitcast_convert_type(lax.shift_left(g, 16), jnp.float32)
        plsc.addupdate_scatter(outs[2 * t], [d_idx], hi * w)
        plsc.addupdate_scatter(outs[2 * t + 1], [d_idx], lo * w)

  def start_chunk(egrp, k, src, dst, w, buf):
    off = egrp * e_grp + k * _CHUNK
    sv, dv, wv, sem = buf
    return (
        pltpu.async_copy(src.at[pl.ds(off, _CHUNK)], sv, sem),
        pltpu.async_copy(dst.at[pl.ds(off, _CHUNK)], dv, sem),
        pltpu.async_copy(w.at[pl.ds(off, _CHUNK)], wv, sem),
    )

  def body(hp, src, dst, w, out, *scratch):
    tabs = scratch[0:n_pk]
    outs = scratch[n_pk:n_pk + dims]
    base = n_pk + dims
    bufs = (scratch[base:base + 4], scratch[base + 4:base + 8])
    tile = lax.axis_index("c") * 16 + lax.axis_index("s")
    egrp = tile // tiles_per_group
    pbase = (tile % tiles_per_group) * n_pk
    for t in range(n_pk):
      pltpu.sync_copy(hp.at[pbase + t], tabs[t])
    zero_columns(outs)
    descs = start_chunk(egrp, 0, src, dst, w, bufs[0])
    for k in range(n_chunks):
      for d in descs:
        d.wait()
      cur = bufs[k % 2]
      if k + 1 < n_chunks:
        descs = start_chunk(egrp, k + 1, src, dst, w, bufs[(k + 1) % 2])
      edge_loop(tabs, outs, cur[0], cur[1], cur[2])
    nf2 = n_feat // 2
    for t in range(n_pk):
      pltpu.sync_copy(outs[2 * t], out.at[egrp, pbase + t])
      pltpu.sync_copy(outs[2 * t + 1], out.at[egrp, nf2 + pbase + t])

  mesh = plsc.VectorSubcoreMesh(core_axis_name="c", subcore_axis_name="s")
  return pl.kernel(
      body,
      out_type=jax.ShapeDtypeStruct((n_groups, n_feat, N_NODES), jnp.float32),
      mesh=mesh,
      compiler_params=pltpu.CompilerParams(needs_layout_passes=False),
      scratch_types=[
          *[pltpu.VMEM((N_NODES,), jnp.int32) for _ in range(n_pk)],
          *[pltpu.VMEM((N_NODES,), jnp.float32) for _ in range(dims)],
          pltpu.VMEM((_CHUNK,), jnp.int32),
          pltpu.VMEM((_CHUNK,), jnp.int32),
          pltpu.VMEM((_CHUNK,), jnp.float32),
          pltpu.SemaphoreType.DMA,
          pltpu.VMEM((_CHUNK,), jnp.int32),
          pltpu.VMEM((_CHUNK,), jnp.int32),
          pltpu.VMEM((_CHUNK,), jnp.float32),
          pltpu.SemaphoreType.DMA,
      ],
  )


def _mm1_body(x_ref, w_ref, out_ref):
  h = lax.dot_general(
      w_ref[...], x_ref[...], (((0,), (1,)), ((), ())),
      preferred_element_type=jnp.float32)
  out_ref[...] = _pack_pairs(h)


def _mid_body(p_ref, b1_ref, w2_ref, out_ref):
  h = p_ref[0] + p_ref[1] + p_ref[2] + p_ref[3]
  h = jnp.maximum(h + b1_ref[...], 0.0)
  g = lax.dot_general(
      w2_ref[...], h, (((0,), (0,)), ((), ())),
      preferred_element_type=jnp.float32)
  out_ref[...] = _pack_pairs(g)


def _final_body(p_ref, b2_ref, eye_ref, out_ref):
  g = p_ref[0]
  for i in range(1, p_ref.shape[0]):
    g = g + p_ref[i]
  logits = lax.dot_general(
      g, eye_ref[...], (((0,), (0,)), ((), ())),
      preferred_element_type=jnp.float32)
  z = logits + b2_ref[0][None, :]
  m = jnp.max(z, axis=1, keepdims=True)
  lse = jnp.log(jnp.sum(jnp.exp(z - m), axis=1, keepdims=True)) + m
  out_ref[...] = z - lse


@jax.jit
def kernel(x, edge_index, edge_weight, W1, b1, W2, b2):
  src = edge_index[0].astype(jnp.int32)
  dst = edge_index[1].astype(jnp.int32)
  w = edge_weight.astype(jnp.float32)

  hp = pl.pallas_call(
      _mm1_body,
      out_shape=jax.ShapeDtypeStruct((D_HID // 2, N_NODES), jnp.int32),
  )(x, W1)

  p1 = _make_sc_propagate(D_HID, 4)(hp, src, dst, w)

  gp = pl.pallas_call(
      _mid_body,
      out_shape=jax.ShapeDtypeStruct((N_CLASSES // 2, N_NODES), jnp.int32),
  )(p1, b1.reshape(D_HID, 1), W2)

  p2 = _make_sc_propagate(N_CLASSES, 4)(gp, src, dst, w)

  out = pl.pallas_call(
      _final_body,
      out_shape=jax.ShapeDtypeStruct((N_NODES, N_CLASSES), jnp.float32),
  )(p2, b2.reshape(1, N_CLASSES), jnp.eye(N_CLASSES, dtype=jnp.float32))
  return out


# speedup vs baseline: 1.1414x; 1.0001x over previous
import jax
import jax.numpy as jnp
from jax import lax
from jax.experimental import pallas as pl
from jax.experimental.pallas import tpu as pltpu
from jax.experimental.pallas import tpu_sc as plsc

N_NODES = 10000
N_EDGES = 320000
D_IN = 128
D_HID = 32
N_CLASSES = 16

_LANES = 16
_N_TILES = 32

_HI_MASK = -65536


def _pack_pairs(h):
  k = h.shape[0] // 2
  r = lax.bitcast_convert_type(h, jnp.int32) + jnp.int32(0x8000)
  hi = jnp.bitwise_and(r[:k], jnp.int32(_HI_MASK))
  lo = lax.shift_right_logical(r[k:], 16)
  return jnp.bitwise_or(hi, lo)


def _make_sc_propagate(n_feat, dims, chunk):
  n_pk = dims // 2
  tiles_per_group = n_feat // dims
  n_groups = _N_TILES // tiles_per_group
  e_grp = N_EDGES // n_groups
  n_chunks = e_grp // chunk
  assert n_chunks * chunk == e_grp

  def zero_columns(outs):
    zero = jnp.zeros((_LANES,), jnp.float32)

    @plsc.parallel_loop(0, N_NODES, _LANES)
    def _(i):
      sl = pl.ds(i, _LANES)
      for o in outs:
        o[sl] = zero

  def edge_loop(tabs, outs, sv, dv, wv):
    @plsc.parallel_loop(0, chunk, _LANES, unroll=4)
    def _(i):
      sl = pl.ds(i, _LANES)
      s_idx = sv[sl]
      d_idx = dv[sl]
      w = wv[sl]
      for t in range(n_pk):
        g = plsc.load_gather(tabs[t], [s_idx])
        hi = lax.bitcast_convert_type(jnp.bitwise_and(g, jnp.int32(_HI_MASK)),
                                      jnp.float32)
        lo = lax.bitcast_convert_type(lax.shift_left(g, 16), jnp.float32)
        plsc.addupdate_scatter(outs[2 * t], [d_idx], hi * w)
        plsc.addupdate_scatter(outs[2 * t + 1], [d_idx], lo * w)

  def start_chunk(egrp, k, src, dst, w, buf):
    off = egrp * e_grp + k * chunk
    sv, dv, wv, sem = buf
    return (
        pltpu.async_copy(src.at[pl.ds(off, chunk)], sv, sem),
        pltpu.async_copy(dst.at[pl.ds(off, chunk)], dv, sem),
        pltpu.async_copy(w.at[pl.ds(off, chunk)], wv, sem),
    )

  def body(hp, src, dst, w, out, *scratch):
    tabs = scratch[0:n_pk]
    outs = scratch[n_pk:n_pk + dims]
    base = n_pk + dims
    bufs = (scratch[base:base + 4], scratch[base + 4:base + 8])
    tile = lax.axis_index("c") * 16 + lax.axis_index("s")
    egrp = tile // tiles_per_group
    pbase = (tile % tiles_per_group) * n_pk
    for t in range(n_pk):
      pltpu.sync_copy(hp.at[pbase + t], tabs[t])
    zero_columns(outs)
    descs = start_chunk(egrp, 0, src, dst, w, bufs[0])
    for k in range(n_chunks):
      for d in descs:
        d.wait()
      cur = bufs[k % 2]
      if k + 1 < n_chunks:
        descs = start_chunk(egrp, k + 1, src, dst, w, bufs[(k + 1) % 2])
      edge_loop(tabs, outs, cur[0], cur[1], cur[2])
    nf2 = n_feat // 2
    for t in range(n_pk):
      pltpu.sync_copy(outs[2 * t], out.at[egrp, pbase + t])
      pltpu.sync_copy(outs[2 * t + 1], out.at[egrp, nf2 + pbase + t])

  mesh = plsc.VectorSubcoreMesh(core_axis_name="c", subcore_axis_name="s")
  return pl.kernel(
      body,
      out_type=jax.ShapeDtypeStruct((n_groups, n_feat, N_NODES), jnp.float32),
      mesh=mesh,
      compiler_params=pltpu.CompilerParams(needs_layout_passes=False),
      scratch_types=[
          *[pltpu.VMEM((N_NODES,), jnp.int32) for _ in range(n_pk)],
          *[pltpu.VMEM((N_NODES,), jnp.float32) for _ in range(dims)],
          pltpu.VMEM((chunk,), jnp.int32),
          pltpu.VMEM((chunk,), jnp.int32),
          pltpu.VMEM((chunk,), jnp.float32),
          pltpu.SemaphoreType.DMA,
          pltpu.VMEM((chunk,), jnp.int32),
          pltpu.VMEM((chunk,), jnp.int32),
          pltpu.VMEM((chunk,), jnp.float32),
          pltpu.SemaphoreType.DMA,
      ],
  )


def _mm1_body(x_ref, w_ref, out_ref):
  h = lax.dot_general(
      w_ref[...], x_ref[...], (((0,), (1,)), ((), ())),
      preferred_element_type=jnp.float32)
  out_ref[...] = _pack_pairs(h)


def _mid_body(p_ref, b1_ref, w2_ref, out_ref):
  h = p_ref[0]
  for i in range(1, p_ref.shape[0]):
    h = h + p_ref[i]
  h = jnp.maximum(h + b1_ref[...], 0.0)
  g = lax.dot_general(
      w2_ref[...], h, (((0,), (0,)), ((), ())),
      preferred_element_type=jnp.float32)
  out_ref[...] = _pack_pairs(g)


def _final_body(p_ref, b2_ref, eye_ref, out_ref):
  g = p_ref[0]
  for i in range(1, p_ref.shape[0]):
    g = g + p_ref[i]
  logits = lax.dot_general(
      g, eye_ref[...], (((0,), (0,)), ((), ())),
      preferred_element_type=jnp.float32)
  z = logits + b2_ref[0][None, :]
  m = jnp.max(z, axis=1, keepdims=True)
  lse = jnp.log(jnp.sum(jnp.exp(z - m), axis=1, keepdims=True)) + m
  out_ref[...] = z - lse


@jax.jit
def kernel(x, edge_index, edge_weight, W1, b1, W2, b2):
  src = edge_index[0].astype(jnp.int32)
  dst = edge_index[1].astype(jnp.int32)
  w = edge_weight.astype(jnp.float32)

  hp = pl.pallas_call(
      _mm1_body,
      out_shape=jax.ShapeDtypeStruct((D_HID // 2, N_NODES), jnp.int32),
  )(x, W1)

  p1 = _make_sc_propagate(D_HID, 4, 4000)(hp, src, dst, w)

  gp = pl.pallas_call(
      _mid_body,
      out_shape=jax.ShapeDtypeStruct((N_CLASSES // 2, N_NODES), jnp.int32),
  )(p1, b1.reshape(D_HID, 1), W2)

  p2 = _make_sc_propagate(N_CLASSES, 4, 4000)(gp, src, dst, w)

  out = pl.pallas_call(
      _final_body,
      out_shape=jax.ShapeDtypeStruct((N_NODES, N_CLASSES), jnp.float32),
  )(p2, b2.reshape(1, N_CLASSES), jnp.eye(N_CLASSES, dtype=jnp.float32))
  return out
